# baseline (speedup 1.0000x reference)
import functools

import jax
import jax.numpy as jnp
from jax import lax
from jax.experimental import pallas as pl
from jax.experimental.pallas import tpu as pltpu
from jax.experimental.pallas import tpu_sc as plsc

_NS = 16
_L = 16


def _make_lookup(table_n: int, batch: int):
  b_per_w = batch // _NS
  half = b_per_w // 2
  mesh = plsc.VectorSubcoreMesh(
      core_axis_name="c", subcore_axis_name="s", num_cores=1)

  @functools.partial(
      pl.kernel,
      mesh=mesh,
      out_type=jax.ShapeDtypeStruct((batch,), jnp.float32),
      scratch_types=[
          pltpu.VMEM((table_n,), jnp.float32),
          pltpu.VMEM((b_per_w,), jnp.int32),
          pltpu.VMEM((b_per_w,), jnp.float32),
          pltpu.SemaphoreType.DMA,
          pltpu.SemaphoreType.DMA,
          pltpu.SemaphoreType.DMA,
          pltpu.SemaphoreType.DMA,
          pltpu.SemaphoreType.DMA,
      ],
      compiler_params=pltpu.CompilerParams(
          needs_layout_passes=False,
          disable_bounds_checks=True,
          disable_semaphore_checks=True,
          skip_device_barrier=True,
      ),
  )
  def lookup(w_hbm, idx_hbm, out_hbm, w_v, idx_v, out_v, sem_w, *sems):
    base = lax.axis_index("s") * b_per_w
    q = b_per_w // 4
    cp_w = pltpu.async_copy(w_hbm, w_v, sem_w)
    cp_in = [
        pltpu.async_copy(
            idx_hbm.at[pl.ds(base + c * q, q)], idx_v.at[pl.ds(c * q, q)],
            sems[c])
        for c in range(4)
    ]
    cp_w.wait()
    cp_out = []
    for c in range(4):
      cp_in[c].wait()

      @plsc.parallel_loop(c * q, (c + 1) * q, step=_L, unroll=8)
      def _(i):
        out_v[pl.ds(i, _L)] = plsc.load_gather(w_v, [idx_v[pl.ds(i, _L)]])

      cp_out.append(
          pltpu.async_copy(
              out_v.at[pl.ds(c * q, q)], out_hbm.at[pl.ds(base + c * q, q)],
              sems[c]))
    for c in range(4):
      cp_out[c].wait()

  return lookup


def kernel(layer_weights, iteration):
  idx = iteration.astype(jnp.int32)
  lookup = _make_lookup(layer_weights.shape[0], idx.shape[0])
  return lookup(layer_weights.astype(jnp.float32), idx)


# speedup vs baseline: 1.0123x; 1.0123x over previous
import functools

import jax
import jax.numpy as jnp
from jax import lax
from jax.experimental import pallas as pl
from jax.experimental.pallas import tpu as pltpu
from jax.experimental.pallas import tpu_sc as plsc

_NS = 16
_L = 16


def _make_lookup(table_n: int, batch: int):
  b_per_w = batch // _NS
  half = b_per_w // 2
  mesh = plsc.VectorSubcoreMesh(
      core_axis_name="c", subcore_axis_name="s", num_cores=1)

  @functools.partial(
      pl.kernel,
      mesh=mesh,
      out_type=jax.ShapeDtypeStruct((batch,), jnp.float32),
      scratch_types=[
          pltpu.VMEM((table_n,), jnp.float32),
          pltpu.VMEM((b_per_w,), jnp.int32),
          pltpu.VMEM((b_per_w,), jnp.float32),
          pltpu.SemaphoreType.DMA,
          pltpu.SemaphoreType.DMA,
          pltpu.SemaphoreType.DMA,
      ],
      compiler_params=pltpu.CompilerParams(
          needs_layout_passes=False,
          disable_bounds_checks=True,
          disable_semaphore_checks=True,
          skip_device_barrier=True,
      ),
  )
  def lookup(w_hbm, idx_hbm, out_hbm, w_v, idx_v, out_v, sem_w, sem_a, sem_b):
    base = lax.axis_index("s") * b_per_w
    cp_w = pltpu.async_copy(w_hbm, w_v, sem_w)
    cp_a = pltpu.async_copy(
        idx_hbm.at[pl.ds(base, half)], idx_v.at[pl.ds(0, half)], sem_a)
    cp_b = pltpu.async_copy(
        idx_hbm.at[pl.ds(base + half, half)], idx_v.at[pl.ds(half, half)],
        sem_b)
    cp_w.wait()
    cp_a.wait()

    @plsc.parallel_loop(0, half, step=_L, unroll=8)
    def _(i):
      out_v[pl.ds(i, _L)] = plsc.load_gather(w_v, [idx_v[pl.ds(i, _L)]])

    cp_oa = pltpu.async_copy(
        out_v.at[pl.ds(0, half)], out_hbm.at[pl.ds(base, half)], sem_a)
    cp_b.wait()

    @plsc.parallel_loop(half, b_per_w, step=_L, unroll=8)
    def _(i):
      out_v[pl.ds(i, _L)] = plsc.load_gather(w_v, [idx_v[pl.ds(i, _L)]])

    cp_ob = pltpu.async_copy(
        out_v.at[pl.ds(half, half)], out_hbm.at[pl.ds(base + half, half)],
        sem_b)
    cp_oa.wait()
    cp_ob.wait()

  return lookup


def kernel(layer_weights, iteration):
  idx = iteration.astype(jnp.int32)
  lookup = _make_lookup(layer_weights.shape[0], idx.shape[0])
  return lookup(layer_weights.astype(jnp.float32), idx)
